# baseline (speedup 1.0000x reference)
import jax
import jax.numpy as jnp
from jax import lax
from jax.experimental import pallas as pl
from jax.experimental.pallas import tpu as pltpu
from jax.experimental.pallas import tpu_sc as plsc

_N_NODES = 100000
_P = 800000
_STENCIL = 8
_DIM = 3
_M = _P * _STENCIL
_M3 = _M * _DIM

_NC = 2
_NS = 16
_NW = _NC * _NS

_CHUNK = 128
_NCHUNKS = _M3 // _CHUNK
_N_ACC = ((_N_NODES * _DIM) + 383) // 384 * 384
_SEG = _N_ACC // _NS


_BP = 4000


def _mul_body(m_ref, g_ref, i_ref, d_ref, o_ref, e_ref):
    o_ref[...] = g_ref[...] * m_ref[...]
    e_ref[...] = jnp.repeat(i_ref[...] * 3, 3, axis=1) + d_ref[...]


def _mul(mass_2d, grad24, ids8, d24):
    grid = _P // _BP
    return pl.pallas_call(
        _mul_body,
        grid=(grid,),
        in_specs=[
            pl.BlockSpec((_BP, 1), lambda i: (i, 0)),
            pl.BlockSpec((_BP, 24), lambda i: (i, 0)),
            pl.BlockSpec((_BP, 8), lambda i: (i, 0)),
            pl.BlockSpec((1, 24), lambda i: (0, 0)),
        ],
        out_specs=[
            pl.BlockSpec((_BP, 24), lambda i: (i, 0)),
            pl.BlockSpec((_BP, 24), lambda i: (i, 0)),
        ],
        out_shape=[
            jax.ShapeDtypeStruct((_P, 24), jnp.float32),
            jax.ShapeDtypeStruct((_P, 24), jnp.int32),
        ],
    )(mass_2d, grad24, ids8, d24)


_MROWS = 8
_MACRO = _MROWS * 128
_NMACROS = _M3 // _MACRO
_NROWS = _M3 // 128


def _sc_body(upd_hbm, eid_hbm, zeros_hbm, out_hbm, updb0, updb1, idxb0,
             idxb1, zbuf, acc, semu, semi):
    c = lax.axis_index("c")
    s = lax.axis_index("s")
    w = c * _NS + s

    seg = s * _SEG
    pltpu.sync_copy(zeros_hbm.at[pl.ds(seg, _SEG)], zbuf)
    pltpu.sync_copy(zbuf, acc.at[pl.ds(seg, _SEG)])
    plsc.subcore_barrier()

    nbase = _NMACROS // _NW
    nextra = _NMACROS % _NW
    nmac = jnp.where(w < nextra, nbase + 1, nbase)
    base = w * nbase + jnp.minimum(w, nextra)

    bufs = ((updb0, idxb0), (updb1, idxb1))

    def copies(b, i):
        el = (base + i) * _MACRO
        ub, ib = bufs[b]
        return (
            pltpu.make_async_copy(eid_hbm.at[pl.ds(el, _MACRO)],
                                  ib, semi.at[b]),
            pltpu.make_async_copy(upd_hbm.at[pl.ds(el, _MACRO)],
                                  ub, semu.at[b]),
        )

    def start(b, i):
        for cp in copies(b, i):
            cp.start()

    def finish(b, i):
        for cp in copies(b, i):
            cp.wait()
        ub, ib = bufs[b]
        pltpu.sync_copy(ub, acc.at[ib], add=True)

    start(0, 0)

    @pl.loop(0, nmac // 2)
    def _(p):
        i0 = 2 * p
        start(1, i0 + 1)
        finish(0, i0)

        @pl.when(i0 + 2 < nmac)
        def _():
            start(0, i0 + 2)

        finish(1, i0 + 1)

    @pl.when(nmac % 2 == 1)
    def _():
        finish(0, nmac - 1)

    plsc.subcore_barrier()
    pltpu.sync_copy(acc.at[pl.ds(seg, _SEG)], zbuf)
    pltpu.sync_copy(zbuf, out_hbm.at[pl.ds(c * _N_ACC + seg, _SEG)])


def _sc_scatter(upd, eids, zeros):
    mesh = plsc.VectorSubcoreMesh(core_axis_name="c", subcore_axis_name="s")
    f = pl.kernel(
        _sc_body,
        out_type=jax.ShapeDtypeStruct((_NC * _N_ACC,), jnp.float32),
        mesh=mesh,
        compiler_params=pltpu.CompilerParams(use_tc_tiling_on_sc=True),
        scratch_types=[
            pltpu.VMEM((_MACRO,), jnp.float32),
            pltpu.VMEM((_MACRO,), jnp.float32),
            pltpu.VMEM((_MACRO,), jnp.int32),
            pltpu.VMEM((_MACRO,), jnp.int32),
            pltpu.VMEM((_SEG,), jnp.float32),
            pltpu.VMEM_SHARED((_N_ACC,), jnp.float32),
            pltpu.SemaphoreType.DMA((2,)),
            pltpu.SemaphoreType.DMA((2,)),
        ],
    )
    return f(upd, eids, zeros)


def _add_body(p_ref, o_ref):
    o_ref[...] = p_ref[0] + p_ref[1]


def _add(partials):
    bn = 2000
    grid = _N_NODES // bn
    return pl.pallas_call(
        _add_body,
        grid=(grid,),
        in_specs=[pl.BlockSpec((2, bn, _DIM), lambda i: (0, i, 0))],
        out_specs=pl.BlockSpec((bn, _DIM), lambda i: (i, 0)),
        out_shape=jax.ShapeDtypeStruct((_N_NODES, _DIM), jnp.float32),
    )(partials)


def kernel(mass_stack, shapef_grad_stack, node_id_stack):
    d24 = jnp.tile(jnp.arange(_DIM, dtype=jnp.int32), _STENCIL)
    contrib24, eids24 = _mul(mass_stack.reshape(_P, 1),
                             shapef_grad_stack.reshape(_P, 24),
                             node_id_stack, d24.reshape(1, 24))
    upd = contrib24.reshape(_M3)
    eids = eids24.reshape(_M3)
    zeros = jnp.zeros((_N_ACC,), jnp.float32)
    partials = _sc_scatter(upd, eids, zeros)
    partials = partials.reshape(_NC, _N_ACC // _DIM, _DIM)
    return _add(partials)


# speedup vs baseline: 1.7930x; 1.7930x over previous
import dataclasses

import jax
import jax.numpy as jnp
from jax import lax
from jax.experimental import pallas as pl
from jax.experimental.pallas import tpu as pltpu
from jax.experimental.pallas import tpu_sc as plsc

_N_NODES = 100000
_P = 800000
_STENCIL = 8
_DIM = 3
_M = _P * _STENCIL
_M3 = _M * _DIM

_NC = 2
_NS = 16
_NW = _NC * _NS

_CHUNK = 128
_NCHUNKS = _M3 // _CHUNK
_N_ACC = ((_N_NODES * _DIM) + 383) // 384 * 384
_SEG = _N_ACC // _NS


_BP = 4000


def _mul_body(m_ref, g_ref, o_ref):
    o_ref[...] = g_ref[...] * m_ref[...]


def _mul(mass_2d, grad24):
    grid = _P // _BP
    return pl.pallas_call(
        _mul_body,
        grid=(grid,),
        in_specs=[
            pl.BlockSpec((_BP, 1), lambda i: (i, 0)),
            pl.BlockSpec((_BP, 24), lambda i: (i, 0)),
        ],
        out_specs=pl.BlockSpec((_BP, 24), lambda i: (i, 0)),
        out_shape=jax.ShapeDtypeStruct((_P, 24), jnp.float32),
    )(mass_2d, grad24)


_MACRO_I = 1024
_MACRO = _MACRO_I * _DIM
_NMACROS = _M // _MACRO_I


def _sc_body(upd_hbm, ids_hbm, zeros_hbm, out_hbm, updb0, updb1, idsb0,
             idsb1, ebuf0, ebuf1, zbuf, acc, semu, semi):
    c = lax.axis_index("c")
    s = lax.axis_index("s")
    w = c * _NS + s

    seg = s * _SEG
    pltpu.sync_copy(zeros_hbm.at[pl.ds(seg, _SEG)], zbuf)
    pltpu.sync_copy(zbuf, acc.at[pl.ds(seg, _SEG)])
    plsc.subcore_barrier()

    nbase = _NMACROS // _NW
    nextra = _NMACROS % _NW
    nmac = jnp.where(w < nextra, nbase + 1, nbase)
    base = w * nbase + jnp.minimum(w, nextra)

    bufs = ((updb0, idsb0, ebuf0), (updb1, idsb1, ebuf1))
    lane3 = lax.iota(jnp.int32, 16) * 3

    def copies(b, i):
        ub, ib = bufs[b][0], bufs[b][1]
        return (
            pltpu.make_async_copy(ids_hbm.at[pl.ds((base + i) * _MACRO_I,
                                                   _MACRO_I)],
                                  ib, semi.at[b]),
            pltpu.make_async_copy(upd_hbm.at[pl.ds((base + i) * _MACRO,
                                                   _MACRO)],
                                  ub, semu.at[b]),
        )

    def start(b, i):
        for cp in copies(b, i):
            cp.start()

    def finish(b, i):
        for cp in copies(b, i):
            cp.wait()
        ub, ib, eb = bufs[b]

        @pl.loop(0, _MACRO_I // 16)
        def _(k):
            v3 = ib[pl.ds(k * 16, 16)] * 3
            off = k * 48
            for d in range(3):
                plsc.store_scatter(eb, [lane3 + (off + d)], v3 + d)

        pltpu.sync_copy(ub, acc.at[eb], add=True)

    start(0, 0)

    @pl.loop(0, nmac // 2)
    def _(p):
        i0 = 2 * p
        start(1, i0 + 1)
        finish(0, i0)

        @pl.when(i0 + 2 < nmac)
        def _():
            start(0, i0 + 2)

        finish(1, i0 + 1)

    @pl.when(nmac % 2 == 1)
    def _():
        finish(0, nmac - 1)

    plsc.subcore_barrier()
    pltpu.sync_copy(acc.at[pl.ds(seg, _SEG)], zbuf)
    pltpu.sync_copy(zbuf, out_hbm.at[pl.ds(c * _N_ACC + seg, _SEG)])


def _sc_compiler_params():
    cp = pltpu.CompilerParams()
    if "needs_layout_passes" in pltpu.CompilerParams.__dataclass_fields__:
        cp = dataclasses.replace(cp, needs_layout_passes=False)
    return cp


def _sc_scatter(upd, ids_flat, zeros):
    mesh = plsc.VectorSubcoreMesh(core_axis_name="c", subcore_axis_name="s")
    f = pl.kernel(
        _sc_body,
        out_type=jax.ShapeDtypeStruct((_NC * _N_ACC,), jnp.float32),
        mesh=mesh,
        compiler_params=_sc_compiler_params(),
        scratch_types=[
            pltpu.VMEM((_MACRO,), jnp.float32),
            pltpu.VMEM((_MACRO,), jnp.float32),
            pltpu.VMEM((_MACRO_I,), jnp.int32),
            pltpu.VMEM((_MACRO_I,), jnp.int32),
            pltpu.VMEM((_MACRO,), jnp.int32),
            pltpu.VMEM((_MACRO,), jnp.int32),
            pltpu.VMEM((_SEG,), jnp.float32),
            pltpu.VMEM_SHARED((_N_ACC,), jnp.float32),
            pltpu.SemaphoreType.DMA((2,)),
            pltpu.SemaphoreType.DMA((2,)),
        ],
    )
    return f(upd, ids_flat, zeros)


def _add_body(p_ref, o_ref):
    o_ref[...] = p_ref[0] + p_ref[1]


def _add(partials):
    bn = 2000
    grid = _N_NODES // bn
    return pl.pallas_call(
        _add_body,
        grid=(grid,),
        in_specs=[pl.BlockSpec((2, bn, _DIM), lambda i: (0, i, 0))],
        out_specs=pl.BlockSpec((bn, _DIM), lambda i: (i, 0)),
        out_shape=jax.ShapeDtypeStruct((_N_NODES, _DIM), jnp.float32),
    )(partials)


def kernel(mass_stack, shapef_grad_stack, node_id_stack):
    contrib24 = _mul(mass_stack.reshape(_P, 1),
                     shapef_grad_stack.reshape(_P, 24))
    upd = contrib24.reshape(_M3)
    ids_flat = node_id_stack.reshape(_M)
    zeros = jnp.zeros((_N_ACC,), jnp.float32)
    partials = _sc_scatter(upd, ids_flat, zeros)
    partials = partials.reshape(_NC, _N_ACC // _DIM, _DIM)
    return _add(partials)
